# baseline (speedup 1.0000x reference)
import functools
import math

import jax
import jax.numpy as jnp
from jax import lax
from jax.experimental import pallas as pl
from jax.experimental.pallas import tpu as pltpu
from jax.experimental.pallas import tpu_sc as plsc

_D = 32
_SCALE = math.sqrt(32.0)
_NC, _NS = 2, 16
_NW = _NC * _NS
_BT = 128
_FT = _D // 8


@functools.lru_cache(maxsize=None)
def _build(n_rows: int, n_cols: int):
    nbt = n_rows // _BT
    bt_per_w = nbt // _NW
    nunit = n_cols * bt_per_w
    assert nbt * _BT == n_rows and bt_per_w * _NW == nbt and nunit % 2 == 0

    mesh = plsc.VectorSubcoreMesh(core_axis_name="c", subcore_axis_name="s")

    @functools.partial(
        pl.kernel,
        out_type=jax.ShapeDtypeStruct(
            (n_cols, _FT, nbt, 8 * _BT), jnp.float32
        ),
        mesh=mesh,
        scratch_types=[
            pltpu.VMEM((n_cols, bt_per_w, _BT), jnp.int32),
            pltpu.VMEM((2, _BT, _D), jnp.float32),
            pltpu.VMEM((_D * _BT,), jnp.float32),
            pltpu.VMEM((_D * _BT,), jnp.float32),
            pltpu.SemaphoreType.DMA,
            pltpu.SemaphoreType.DMA,
            pltpu.SemaphoreType.DMA,
            pltpu.SemaphoreType.DMA,
        ],
        compiler_params=pltpu.CompilerParams(
            use_tc_tiling_on_sc=False,
            needs_layout_passes=False,
            disable_bounds_checks=True,
        ),
    )
    def emb(tok_hbm, table_hbm, out_hbm, idx_v, rows_v, s0, s1,
            g0, g1, o0, o1):
        stages = (s0, s1)
        w = lax.axis_index("s") * _NC + lax.axis_index("c")
        bt0 = w * bt_per_w
        gsems = (g0, g1)
        osems = (o0, o1)
        pltpu.sync_copy(tok_hbm.at[:, pl.ds(bt0, bt_per_w)], idx_v)

        lanes = lax.iota(jnp.int32, 16)
        f_vecs = [jnp.full((16,), f, jnp.int32) for f in range(_D)]

        def unit_jbt(t):
            return t // bt_per_w, t % bt_per_w

        def issue_gather(t, p):
            j, btl = unit_jbt(t)
            pltpu.async_copy(
                table_hbm.at[idx_v.at[j, btl]],
                rows_v.at[p],
                gsems[p],
            )

        def drain_gather(p):
            pltpu.make_async_copy(
                table_hbm.at[pl.ds(0, _BT)], rows_v.at[p], gsems[p]
            ).wait()

        def drain_write(p):
            for ft in range(_FT):
                pltpu.make_async_copy(
                    out_hbm.at[0, 0, 0],
                    stages[p].at[pl.ds(ft * 8 * _BT, 8 * _BT)],
                    osems[p],
                ).wait()

        issue_gather(0, 0)

        @pl.loop(0, nunit, step=2)
        def _unit2(t0):
            for dp in range(2):
                t = t0 + dp
                p = dp
                drain_gather(p)

                @pl.when(t + 1 < nunit)
                def _():
                    issue_gather(t + 1, 1 - p)

                @pl.when(t >= 2)
                def _():
                    drain_write(p)

                @pl.loop(0, _BT // 16)
                def _tr(b16):
                    bi_idx = lanes + b16 * 16
                    for f in range(_D):
                        vals = (
                            plsc.load_gather(
                                rows_v.at[p], [bi_idx, f_vecs[f]]
                            )
                            * _SCALE
                        )
                        stages[p][pl.ds(f * _BT + b16 * 16, 16)] = vals

                j, btl = unit_jbt(t)
                for ft in range(_FT):
                    pltpu.async_copy(
                        stages[p].at[pl.ds(ft * 8 * _BT, 8 * _BT)],
                        out_hbm.at[j, ft, bt0 + btl],
                        osems[p],
                    )

        drain_write(0)
        drain_write(1)

    return emb


def kernel(tokens, embedding_weight):
    n_rows, n_cols = tokens.shape
    nbt = n_rows // _BT
    toks = jnp.swapaxes(tokens.astype(jnp.int32), 0, 1).reshape(
        n_cols, nbt, _BT
    )
    phys = _build(n_rows, n_cols)(toks, embedding_weight)
    phys = phys.reshape(n_cols, _FT, nbt, 8, _BT)
    out = jnp.transpose(phys, (2, 4, 0, 1, 3))
    return out.reshape(n_rows, n_cols, _D)


# speedup vs baseline: 1.3620x; 1.3620x over previous
import functools
import math

import jax
import jax.numpy as jnp
from jax import lax
from jax.experimental import pallas as pl
from jax.experimental.pallas import tpu as pltpu
from jax.experimental.pallas import tpu_sc as plsc

_D = 32
_SCALE = math.sqrt(32.0)
_NC, _NS = 2, 16
_NW = _NC * _NS
_BT = 128
_FT = _D // 8


@functools.lru_cache(maxsize=None)
def _build(n_rows: int, n_cols: int):
    nbt = n_rows // _BT
    bt_per_w = nbt // _NW
    nunit = n_cols * bt_per_w
    assert nbt * _BT == n_rows and bt_per_w * _NW == nbt and nunit % 2 == 0

    mesh = plsc.VectorSubcoreMesh(core_axis_name="c", subcore_axis_name="s")

    @functools.partial(
        pl.kernel,
        out_type=jax.ShapeDtypeStruct(
            (n_cols, _FT, nbt, 8 * _BT), jnp.float32
        ),
        mesh=mesh,
        scratch_types=[
            pltpu.VMEM((n_cols, bt_per_w, _BT), jnp.int32),
            pltpu.VMEM((2, _BT, _D), jnp.float32),
            pltpu.VMEM((_D * _BT,), jnp.float32),
            pltpu.VMEM((_D * _BT,), jnp.float32),
            pltpu.SemaphoreType.DMA,
            pltpu.SemaphoreType.DMA,
            pltpu.SemaphoreType.DMA,
            pltpu.SemaphoreType.DMA,
        ],
        compiler_params=pltpu.CompilerParams(
            use_tc_tiling_on_sc=False,
            needs_layout_passes=False,
            disable_bounds_checks=True,
        ),
    )
    def emb(tok_hbm, table_hbm, out_hbm, idx_v, rows_v, s0, s1,
            g0, g1, o0, o1):
        stages = (s0, s1)
        w = lax.axis_index("s") * _NC + lax.axis_index("c")
        bt0 = w * bt_per_w
        gsems = (g0, g1)
        osems = (o0, o1)
        pltpu.sync_copy(tok_hbm.at[:, pl.ds(bt0, bt_per_w)], idx_v)

        lanes = lax.iota(jnp.int32, 16)
        f_vecs = [jnp.full((16,), f, jnp.int32) for f in range(_D)]

        def unit_jbt(t):
            return t // bt_per_w, t % bt_per_w

        def issue_gather(t, p):
            j, btl = unit_jbt(t)
            pltpu.async_copy(
                table_hbm.at[idx_v.at[j, btl]],
                rows_v.at[p],
                gsems[p],
            )

        def drain_gather(p):
            pltpu.make_async_copy(
                table_hbm.at[pl.ds(0, _BT)], rows_v.at[p], gsems[p]
            ).wait()

        def drain_write(p):
            for ft in range(_FT):
                pltpu.make_async_copy(
                    out_hbm.at[0, 0, 0],
                    stages[p].at[pl.ds(ft * 8 * _BT, 8 * _BT)],
                    osems[p],
                ).wait()

        issue_gather(0, 0)

        @pl.loop(0, nunit, step=2)
        def _unit2(t0):
            for dp in range(2):
                t = t0 + dp
                p = dp
                drain_gather(p)

                @pl.when(t + 1 < nunit)
                def _():
                    issue_gather(t + 1, 1 - p)

                @pl.when(t >= 2)
                def _():
                    drain_write(p)

                @pl.loop(0, _BT // 16)
                def _tr(b16):
                    bi_idx = lanes + b16 * 16
                    vals = [
                        plsc.load_gather(rows_v.at[p], [bi_idx, f_vecs[f]])
                        * _SCALE
                        for f in range(_D)
                    ]
                    for f in range(_D):
                        stages[p][pl.ds(f * _BT + b16 * 16, 16)] = vals[f]

                j, btl = unit_jbt(t)
                for ft in range(_FT):
                    pltpu.async_copy(
                        stages[p].at[pl.ds(ft * 8 * _BT, 8 * _BT)],
                        out_hbm.at[j, ft, bt0 + btl],
                        osems[p],
                    )

        drain_write(0)
        drain_write(1)

    return emb


def kernel(tokens, embedding_weight):
    n_rows, n_cols = tokens.shape
    nbt = n_rows // _BT
    toks = jnp.swapaxes(tokens.astype(jnp.int32), 0, 1).reshape(
        n_cols, nbt, _BT
    )
    phys = _build(n_rows, n_cols)(toks, embedding_weight)
    phys = phys.reshape(n_cols, _FT, nbt, 8, _BT)
    out = jnp.transpose(phys, (2, 4, 0, 1, 3))
    return out.reshape(n_rows, n_cols, _D)


# speedup vs baseline: 1.3631x; 1.0008x over previous
import functools
import math

import jax
import jax.numpy as jnp
from jax import lax
from jax.experimental import pallas as pl
from jax.experimental.pallas import tpu as pltpu
from jax.experimental.pallas import tpu_sc as plsc

_D = 32
_SCALE = math.sqrt(32.0)
_NC, _NS = 2, 16
_NW = _NC * _NS
_BT = 128
_FT = _D // 8


@functools.lru_cache(maxsize=None)
def _build(n_rows: int, n_cols: int):
    nbt = n_rows // _BT
    bt_per_w = nbt // _NW
    nunit = n_cols * bt_per_w
    assert nbt * _BT == n_rows and bt_per_w * _NW == nbt and nunit % 2 == 0

    mesh = plsc.VectorSubcoreMesh(core_axis_name="c", subcore_axis_name="s")

    @functools.partial(
        pl.kernel,
        out_type=jax.ShapeDtypeStruct(
            (n_cols, _FT, nbt, 8 * _BT), jnp.float32
        ),
        mesh=mesh,
        scratch_types=[
            pltpu.VMEM((n_cols, bt_per_w, _BT), jnp.int32),
            pltpu.VMEM((4, _BT, _D), jnp.float32),
            pltpu.VMEM((_D * _BT,), jnp.float32),
            pltpu.VMEM((_D * _BT,), jnp.float32),
            pltpu.SemaphoreType.DMA,
            pltpu.SemaphoreType.DMA,
            pltpu.SemaphoreType.DMA,
            pltpu.SemaphoreType.DMA,
            pltpu.SemaphoreType.DMA,
            pltpu.SemaphoreType.DMA,
        ],
        compiler_params=pltpu.CompilerParams(
            use_tc_tiling_on_sc=False,
            needs_layout_passes=False,
            disable_bounds_checks=True,
        ),
    )
    def emb(tok_hbm, table_hbm, out_hbm, idx_v, rows_v, s0, s1,
            g0, g1, g2, g3, o0, o1):
        stages = (s0, s1)
        w = lax.axis_index("s") * _NC + lax.axis_index("c")
        bt0 = w * bt_per_w
        gsems = (g0, g1, g2, g3)
        osems = (o0, o1)
        pltpu.sync_copy(tok_hbm.at[:, pl.ds(bt0, bt_per_w)], idx_v)

        lanes = lax.iota(jnp.int32, 16)
        f_vecs = [jnp.full((16,), f, jnp.int32) for f in range(_D)]

        def unit_jbt(t):
            return t // bt_per_w, t % bt_per_w

        def issue_gather(t, p):
            j, btl = unit_jbt(t)
            pltpu.async_copy(
                table_hbm.at[idx_v.at[j, btl]],
                rows_v.at[p],
                gsems[p],
            )

        def drain_gather(p):
            pltpu.make_async_copy(
                table_hbm.at[pl.ds(0, _BT)], rows_v.at[p], gsems[p]
            ).wait()

        def drain_write(p):
            for ft in range(_FT):
                pltpu.make_async_copy(
                    out_hbm.at[0, 0, 0],
                    stages[p].at[pl.ds(ft * 8 * _BT, 8 * _BT)],
                    osems[p],
                ).wait()

        issue_gather(0, 0)
        issue_gather(1, 1)
        issue_gather(2, 2)

        @pl.loop(0, nunit, step=4)
        def _unit4(t0):
            for dp in range(4):
                t = t0 + dp
                p = dp
                q = dp % 2
                drain_gather(p)

                @pl.when(t + 3 < nunit)
                def _():
                    issue_gather(t + 3, (p + 3) % 4)

                @pl.when(t >= 2)
                def _():
                    drain_write(q)

                @pl.loop(0, _BT // 16)
                def _tr(b16):
                    bi_idx = lanes + b16 * 16
                    vals = [
                        plsc.load_gather(rows_v.at[p], [bi_idx, f_vecs[f]])
                        * _SCALE
                        for f in range(_D)
                    ]
                    for f in range(_D):
                        stages[q][pl.ds(f * _BT + b16 * 16, 16)] = vals[f]

                j, btl = unit_jbt(t)
                for ft in range(_FT):
                    pltpu.async_copy(
                        stages[q].at[pl.ds(ft * 8 * _BT, 8 * _BT)],
                        out_hbm.at[j, ft, bt0 + btl],
                        osems[q],
                    )

        drain_write(0)
        drain_write(1)

    return emb


def kernel(tokens, embedding_weight):
    n_rows, n_cols = tokens.shape
    nbt = n_rows // _BT
    toks = jnp.swapaxes(tokens.astype(jnp.int32), 0, 1).reshape(
        n_cols, nbt, _BT
    )
    phys = _build(n_rows, n_cols)(toks, embedding_weight)
    phys = phys.reshape(n_cols, _FT, nbt, 8, _BT)
    out = jnp.transpose(phys, (2, 4, 0, 1, 3))
    return out.reshape(n_rows, n_cols, _D)
